# baseline (speedup 1.0000x reference)
import jax
import jax.numpy as jnp
from jax import lax
from jax.experimental import pallas as pl
from jax.experimental.pallas import tpu as pltpu
from jax.experimental.pallas import tpu_sc as plsc

DIM = 64
NC = 2
NS = 16
NW = NC * NS
CHUNK = 128


def _gather_body(table_hbm, idx_hbm, out_hbm, idx_v, rows_v, sem):
    wid = lax.axis_index("s") * NC + lax.axis_index("c")
    pltpu.sync_copy(idx_hbm.at[wid], idx_v)
    copies = [
        pltpu.async_copy(table_hbm.at[idx_v.at[j]], rows_v.at[j], sem)
        for j in range(idx_v.shape[0])
    ]
    for c in copies:
        c.wait()
    pltpu.sync_copy(rows_v, out_hbm.at[wid])


def _sc_gather(table, idx):
    batch = idx.shape[0]
    b_per_w = batch // NW
    nchunk = b_per_w // CHUNK
    idx3 = idx.reshape(NW, nchunk, CHUNK)
    fn = pl.kernel(
        _gather_body,
        mesh=plsc.VectorSubcoreMesh(core_axis_name="c", subcore_axis_name="s"),
        compiler_params=pltpu.CompilerParams(use_tc_tiling_on_sc=False),
        out_type=jax.ShapeDtypeStruct((NW, nchunk, CHUNK, DIM), jnp.float32),
        scratch_types=[
            pltpu.VMEM((nchunk, CHUNK), jnp.int32),
            pltpu.VMEM((nchunk, CHUNK, DIM), jnp.float32),
            pltpu.SemaphoreType.DMA,
        ],
    )
    return fn(table, idx3).reshape(batch, DIM)


def _proj_body(h_ref, w_ref, b_ref, o_ref):
    h = h_ref[...]
    y = lax.dot_general(h, w_ref[...], (((1,), (1,)), ((), ())),
                        preferred_element_type=jnp.float32)
    y = y + b_ref[...]
    s = jnp.sum(y * y, axis=1, keepdims=True)
    n = jnp.maximum(jnp.sqrt(s), 1e-12)
    o_ref[...] = y / n


def _proj_norm(h, W, b):
    batch = h.shape[0]
    bm = 2048
    grid = (batch // bm,)
    return pl.pallas_call(
        _proj_body,
        grid=grid,
        in_specs=[
            pl.BlockSpec((bm, DIM), lambda i: (i, 0)),
            pl.BlockSpec((DIM, DIM), lambda i: (0, 0)),
            pl.BlockSpec((1, DIM), lambda i: (0, 0)),
        ],
        out_specs=pl.BlockSpec((bm, DIM), lambda i: (i, 0)),
        out_shape=jax.ShapeDtypeStruct((batch, DIM), jnp.float32),
    )(h, W, b.reshape(1, DIM))


def kernel(item_ids, emb_table, W, b):
    h = _sc_gather(emb_table, item_ids)
    return _proj_norm(h, W, b)


# speedup vs baseline: 4.8385x; 4.8385x over previous
import jax
import jax.numpy as jnp
from jax import lax
from jax.experimental import pallas as pl
from jax.experimental.pallas import tpu as pltpu
from jax.experimental.pallas import tpu_sc as plsc

DIM = 64
NC = 2
NS = 16
NW = NC * NS
FLIGHT = 16


def _gather_body(tbl_ref, idx_ref, out_ref, idx_v, stage_v, h_v, sem):
    wid = lax.axis_index("s") * NC + lax.axis_index("c")
    b_per_w = h_v.shape[1]
    iota = lax.iota(jnp.int32, 16)
    a_vecs = [(16 * k + iota) // 8 for k in range(4)]
    s_vec = iota % 8
    row_vecs = [16 * k + iota for k in range(4)]

    pltpu.sync_copy(idx_ref.at[wid], idx_v)

    def chunk(j, carry):
        vec = idx_v[pl.ds(j * FLIGHT, FLIGHT)]
        cps = []
        for u in range(FLIGHT):
            base = (vec[u] // 16) * 16
            cps.append(pltpu.async_copy(
                tbl_ref.at[:, :, pl.ds(base, 16)],
                stage_v.at[:, :, pl.ds(u * 16, 16)],
                sem,
            ))
        for cp in cps:
            cp.wait()
        for u in range(FLIGHT):
            lane = jnp.full((16,), u * 16 + vec[u] % 16, jnp.int32)
            col = jnp.full((16,), j * FLIGHT + u, jnp.int32)
            for k in range(4):
                vals = plsc.load_gather(stage_v, [a_vecs[k], s_vec, lane])
                plsc.store_scatter(h_v, [row_vecs[k], col], vals)
        return carry

    lax.fori_loop(0, b_per_w // FLIGHT, chunk, 0)
    pltpu.sync_copy(h_v, out_ref.at[:, pl.ds(wid * b_per_w, b_per_w)])


def _sc_gather_t(tbl3, idx):
    batch = idx.shape[0]
    b_per_w = batch // NW
    idx2 = idx.reshape(NW, b_per_w)
    fn = pl.kernel(
        _gather_body,
        mesh=plsc.VectorSubcoreMesh(core_axis_name="c", subcore_axis_name="s"),
        compiler_params=pltpu.CompilerParams(needs_layout_passes=False),
        out_type=jax.ShapeDtypeStruct((DIM, batch), jnp.float32),
        scratch_types=[
            pltpu.VMEM((b_per_w,), jnp.int32),
            pltpu.VMEM((8, 8, 16 * FLIGHT), jnp.float32),
            pltpu.VMEM((DIM, b_per_w), jnp.float32),
            pltpu.SemaphoreType.DMA,
        ],
    )
    return fn(tbl3, idx2)


def _proj_body(h_ref, w_ref, b_ref, o_ref):
    h = h_ref[...]
    y = lax.dot_general(w_ref[...], h, (((1,), (0,)), ((), ())),
                        preferred_element_type=jnp.float32)
    y = y + b_ref[...]
    s = jnp.sum(y * y, axis=0, keepdims=True)
    n = jnp.maximum(jnp.sqrt(s), 1e-12)
    o_ref[...] = y / n


def _proj_norm_t(hT, W, b):
    batch = hT.shape[1]
    bn = 2048
    grid = (batch // bn,)
    return pl.pallas_call(
        _proj_body,
        grid=grid,
        in_specs=[
            pl.BlockSpec((DIM, bn), lambda i: (0, i)),
            pl.BlockSpec((DIM, DIM), lambda i: (0, 0)),
            pl.BlockSpec((DIM, 1), lambda i: (0, 0)),
        ],
        out_specs=pl.BlockSpec((DIM, bn), lambda i: (0, i)),
        out_shape=jax.ShapeDtypeStruct((DIM, batch), jnp.float32),
    )(hT, W, b.reshape(DIM, 1))


def kernel(item_ids, emb_table, W, b):
    batch = item_ids.shape[0]
    tbl3 = emb_table.T.reshape(8, DIM // 8, emb_table.shape[0])
    hT = _sc_gather_t(tbl3, item_ids)
    yT = _proj_norm_t(hT, W, b)
    return yT.T


# speedup vs baseline: 7.3288x; 1.5147x over previous
import jax
import jax.numpy as jnp
from jax import lax
from jax.experimental import pallas as pl
from jax.experimental.pallas import tpu as pltpu
from jax.experimental.pallas import tpu_sc as plsc

DIM = 64
NC = 2
NS = 16
NW = NC * NS
FLIGHT = 16


def _gather_body(tbl_ref, idx_ref, out_ref, idx_v, stage_v, h_v, sem):
    wid = lax.axis_index("s") * NC + lax.axis_index("c")
    b_per_w = h_v.shape[1]
    nchunk = b_per_w // FLIGHT
    iota = lax.iota(jnp.int32, 16)

    pltpu.sync_copy(idx_ref.at[wid], idx_v)

    def fire(j):
        half = (j % 2) * (16 * FLIGHT)
        vec = idx_v[pl.ds(j * FLIGHT, FLIGHT)]
        for u in range(FLIGHT):
            base = (vec[u] // 16) * 16
            pltpu.async_copy(
                tbl_ref.at[:, :, pl.ds(base, 16)],
                stage_v.at[:, :, pl.ds(half + u * 16, 16)],
                sem,
            )

    def drain():
        pltpu.make_async_copy(
            tbl_ref.at[:, :, pl.ds(0, 16 * FLIGHT)],
            stage_v.at[:, :, pl.ds(0, 16 * FLIGHT)],
            sem,
        ).wait()

    fire(0)

    def chunk(j, carry):
        @pl.when(j + 1 < nchunk)
        def _():
            fire(j + 1)
        drain()
        half = (j % 2) * (16 * FLIGHT)
        vec = idx_v[pl.ds(j * FLIGHT, FLIGHT)]
        lanes = half + iota * 16 + vec % 16
        for c in range(DIM):
            vals = plsc.load_gather(
                stage_v, [jnp.full((16,), c // 8, jnp.int32),
                          jnp.full((16,), c % 8, jnp.int32), lanes])
            h_v[c, pl.ds(j * FLIGHT, FLIGHT)] = vals
        return carry

    lax.fori_loop(0, nchunk, chunk, 0)
    pltpu.sync_copy(h_v, out_ref.at[:, pl.ds(wid * b_per_w, b_per_w)])


def _sc_gather_t(tbl3, idx):
    batch = idx.shape[0]
    b_per_w = batch // NW
    idx2 = idx.reshape(NW, b_per_w)
    fn = pl.kernel(
        _gather_body,
        mesh=plsc.VectorSubcoreMesh(core_axis_name="c", subcore_axis_name="s"),
        compiler_params=pltpu.CompilerParams(needs_layout_passes=False),
        out_type=jax.ShapeDtypeStruct((DIM, batch), jnp.float32),
        scratch_types=[
            pltpu.VMEM((b_per_w,), jnp.int32),
            pltpu.VMEM((8, 8, 32 * FLIGHT), jnp.float32),
            pltpu.VMEM((DIM, b_per_w), jnp.float32),
            pltpu.SemaphoreType.DMA,
        ],
    )
    return fn(tbl3, idx2)


def _proj_body(h_ref, w_ref, b_ref, o_ref):
    h = h_ref[...]
    y = lax.dot_general(w_ref[...], h, (((1,), (0,)), ((), ())),
                        preferred_element_type=jnp.float32)
    y = y + b_ref[...]
    s = jnp.sum(y * y, axis=0, keepdims=True)
    n = jnp.maximum(jnp.sqrt(s), 1e-12)
    o_ref[...] = y / n


def _proj_norm_t(hT, W, b):
    batch = hT.shape[1]
    bn = 2048
    grid = (batch // bn,)
    return pl.pallas_call(
        _proj_body,
        grid=grid,
        in_specs=[
            pl.BlockSpec((DIM, bn), lambda i: (0, i)),
            pl.BlockSpec((DIM, DIM), lambda i: (0, 0)),
            pl.BlockSpec((DIM, 1), lambda i: (0, 0)),
        ],
        out_specs=pl.BlockSpec((DIM, bn), lambda i: (0, i)),
        out_shape=jax.ShapeDtypeStruct((DIM, batch), jnp.float32),
    )(hT, W, b.reshape(DIM, 1))


def kernel(item_ids, emb_table, W, b):
    batch = item_ids.shape[0]
    tbl3 = emb_table.T.reshape(8, DIM // 8, emb_table.shape[0])
    hT = _sc_gather_t(tbl3, item_ids)
    yT = _proj_norm_t(hT, W, b)
    return yT.T
